# baseline (speedup 1.0000x reference)
import jax
import jax.numpy as jnp
from jax.experimental import pallas as pl

U = 0.83
NUM_CANDIDATES = 256
TOP_K = 10
KBLK = 2048
NEG = -3.0e38
V8 = (-28.0, -20.0, -12.0, -4.0, 4.0, 12.0, 20.0, 28.0)

INTERPRET = False


def _k1_body(k_ref, out_ref):
    k = k_ref[...]
    r = jnp.sum(k * k, axis=1, keepdims=True)
    m = jnp.max(r, axis=0, keepdims=True)
    out_ref[...] = jnp.broadcast_to(m[None, :, :], out_ref.shape)


def _k2k_body(k_ref, a_ref, inv_ref, out_ref):
    s = inv_ref[0, 0]
    ku = k_ref[...] * s
    n2 = jnp.sum(ku * ku, axis=1, keepdims=True)
    n4 = n2 * n2
    n8 = n4 * n4
    zpad = jnp.zeros((ku.shape[0], 5), jnp.float32)
    p = jnp.concatenate([ku, n2, n4, n8, zpad], axis=1)
    pre = jax.lax.dot_general(
        p, a_ref[...], (((1,), (0,)), ((), ())),
        preferred_element_type=jnp.float32)
    out_ref[...] = jnp.where(pre > 0, 1.0, -1.0).astype(jnp.bfloat16)


def _k2q_body(q_ref, a_ref, inv_ref, out_ref):
    s = inv_ref[0, 0]
    qu = q_ref[...] * s
    halves = jnp.full((qu.shape[0], 3), 0.5, jnp.float32)
    zpad = jnp.zeros((qu.shape[0], 5), jnp.float32)
    p = jnp.concatenate([qu, halves, zpad], axis=1)
    pre = jax.lax.dot_general(
        p, a_ref[...], (((1,), (0,)), ((), ())),
        preferred_element_type=jnp.float32)
    out_ref[...] = jnp.where(pre > 0, 1.0, -1.0).astype(jnp.bfloat16)


def _votes(qc_ref, kc_ref):
    return jax.lax.dot_general(
        qc_ref[...], kc_ref[...], (((1,), (1,)), ((), ())),
        preferred_element_type=jnp.float32)


def _valid_mask(shape, nkeys):
    i = pl.program_id(0)
    kloc = jax.lax.broadcasted_iota(jnp.int32, shape, 1)
    return kloc < (nkeys - i * KBLK)


def _count_dot(inds):
    n = len(inds)
    stacked = jnp.concatenate(inds, axis=1).astype(jnp.bfloat16)
    kb = inds[0].shape[1]
    sel = (jax.lax.broadcasted_iota(jnp.int32, (n * kb, n), 0) // kb
           == jax.lax.broadcasted_iota(jnp.int32, (n * kb, n), 1))
    ones = jnp.where(sel, 1.0, 0.0).astype(jnp.bfloat16)
    return jax.lax.dot_general(
        stacked, ones, (((1,), (0,)), ((), ())),
        preferred_element_type=jnp.float32)


def _k3a_body(qc_ref, kc_ref, out_ref, *, nkeys):
    v = _votes(qc_ref, kc_ref)
    valid = _valid_mask(v.shape, nkeys)
    inds = [jnp.where((v >= vj) & valid, 1.0, 0.0) for vj in V8]
    out_ref[...] = _count_dot(inds)[None, :, :]


def _k3b_body(qc_ref, kc_ref, f0_ref, out_ref, *, nkeys):
    v = _votes(qc_ref, kc_ref)
    valid = _valid_mask(v.shape, nkeys)
    f0 = f0_ref[...]
    inds = [jnp.where((v >= f0 + 2.0 * j) & valid, 1.0, 0.0)
            for j in range(4)]
    out_ref[...] = _count_dot(inds)[None, :, :]


def _k4_body(q_ref, k_ref, qc_ref, kc_ref, t_ref, qr_ref, s_ref, m_ref,
             *, nkeys):
    v = _votes(qc_ref, kc_ref)
    valid = _valid_mask(v.shape, nkeys)
    t = t_ref[...]
    eq = (v == t) & valid
    gt = (v > t) & valid
    eqf = jnp.where(eq, 1.0, 0.0)
    p = eqf
    sh = 1
    while sh < p.shape[1]:
        shifted = jnp.concatenate(
            [jnp.zeros((p.shape[0], sh), jnp.float32), p[:, :-sh]], axis=1)
        p = p + shifted
        sh *= 2
    ex = p - eqf
    qr = qr_ref[0, :, :]
    sel = gt | (eq & (ex < qr))

    s = jax.lax.dot_general(
        q_ref[...], k_ref[...], (((1,), (1,)), ((), ())),
        preferred_element_type=jnp.float32)
    ms = jnp.where(sel, s, NEG)
    s_ref[...] = ms
    cols = [jnp.max(ms[:, c * 128:(c + 1) * 128], axis=1, keepdims=True)
            for c in range(KBLK // 128)]
    m_ref[...] = jnp.concatenate(cols, axis=1)[None, :, :]


def _topchunk_body(m_ref, cv_ref, ci_ref):
    V = m_ref[...]
    I = jax.lax.broadcasted_iota(jnp.int32, V.shape, 1)
    q_n = V.shape[0]
    vals, idxs = [], []
    for _ in range(TOP_K):
        m = jnp.max(V, axis=1, keepdims=True)
        hit = (V == m) & (m > NEG)
        am = jnp.max(jnp.where(hit, I, -1), axis=1, keepdims=True)
        V = jnp.where(I == am, NEG, V)
        vals.append(m)
        idxs.append(am)
    pad = 16 - TOP_K
    vals.append(jnp.full((q_n, pad), NEG, jnp.float32))
    idxs.append(jnp.full((q_n, pad), -1, jnp.int32))
    cv_ref[...] = jnp.concatenate(vals, axis=1)
    ci_ref[...] = jnp.concatenate(idxs, axis=1)


def _final_body(g_ref, gi_ref, val_ref, v_ref, i_ref):
    cur = jnp.where(val_ref[...] > 0, g_ref[...], NEG)
    I = gi_ref[...]
    q_n = cur.shape[0]
    vals, idxs = [], []
    for _ in range(TOP_K):
        m = jnp.max(cur, axis=1, keepdims=True)
        hit = (cur == m) & (m > NEG)
        am = jnp.max(jnp.where(hit, I, -1), axis=1, keepdims=True)
        cur = jnp.where(I == am, NEG, cur)
        vals.append(m)
        idxs.append(am)
    pad = 16 - TOP_K
    vals.append(jnp.full((q_n, pad), NEG, jnp.float32))
    idxs.append(jnp.full((q_n, pad), -1, jnp.int32))
    v_ref[...] = jnp.concatenate(vals, axis=1)
    i_ref[...] = jnp.concatenate(idxs, axis=1)


def _sc_gather_rows(table, idx3, nrows):
    from jax.experimental.pallas import tpu as pltpu
    from jax.experimental.pallas import tpu_sc as plsc
    from jax import lax
    ch = idx3.shape[1]
    bpw = ch * 64
    mesh = plsc.VectorSubcoreMesh(core_axis_name="c", subcore_axis_name="s")

    def body(tab_ref, idx_ref, out_ref, idx_v, rows_v, sem):
        wid = lax.axis_index("s") * 2 + lax.axis_index("c")
        pltpu.sync_copy(idx_ref.at[wid], idx_v)
        for j in range(ch):
            pltpu.async_copy(tab_ref.at[idx_v.at[j]],
                             rows_v.at[pl.ds(j * 64, 64)], sem).wait()
        pltpu.sync_copy(rows_v, out_ref.at[pl.ds(wid * bpw, bpw)])

    return pl.kernel(
        body,
        out_type=jax.ShapeDtypeStruct((nrows, 128), jnp.float32),
        mesh=mesh,
        scratch_types=[
            pltpu.VMEM((ch, 64), jnp.int32),
            pltpu.VMEM((bpw, 128), jnp.float32),
            pltpu.SemaphoreType.DMA,
        ],
    )(table, idx3)


def kernel(queries, keys, a):
    import functools
    K, D = keys.shape
    Q = queries.shape[0]
    H = a.shape[1]
    NB = (K + KBLK - 1) // KBLK
    KP = NB * KBLK

    keys_pad = jnp.concatenate(
        [keys, jnp.zeros((KP - K, D), jnp.float32)], axis=0)
    a_pad = jnp.concatenate([a, jnp.zeros((5, H), jnp.float32)], axis=0)

    rmax = pl.pallas_call(
        _k1_body, grid=(NB,),
        in_specs=[pl.BlockSpec((KBLK, D), lambda i: (i, 0))],
        out_specs=pl.BlockSpec((1, 1, 128), lambda i: (i, 0, 0)),
        out_shape=jax.ShapeDtypeStruct((NB, 1, 128), jnp.float32),
        interpret=INTERPRET,
    )(keys_pad)
    denom = jnp.sqrt(jnp.max(rmax))
    inv = jnp.full((1, 128), U / denom, jnp.float32)

    kc = pl.pallas_call(
        _k2k_body, grid=(NB,),
        in_specs=[
            pl.BlockSpec((KBLK, D), lambda i: (i, 0)),
            pl.BlockSpec((136, H), lambda i: (0, 0)),
            pl.BlockSpec((1, 128), lambda i: (0, 0)),
        ],
        out_specs=pl.BlockSpec((KBLK, H), lambda i: (i, 0)),
        out_shape=jax.ShapeDtypeStruct((KP, H), jnp.bfloat16),
        interpret=INTERPRET,
    )(keys_pad, a_pad, inv)

    qc = pl.pallas_call(
        _k2q_body, grid=(1,),
        in_specs=[
            pl.BlockSpec((Q, D), lambda i: (0, 0)),
            pl.BlockSpec((136, H), lambda i: (0, 0)),
            pl.BlockSpec((1, 128), lambda i: (0, 0)),
        ],
        out_specs=pl.BlockSpec((Q, H), lambda i: (0, 0)),
        out_shape=jax.ShapeDtypeStruct((Q, H), jnp.bfloat16),
        interpret=INTERPRET,
    )(queries, a_pad, inv)

    C8 = pl.pallas_call(
        functools.partial(_k3a_body, nkeys=K), grid=(NB,),
        in_specs=[
            pl.BlockSpec((Q, H), lambda i: (0, 0)),
            pl.BlockSpec((KBLK, H), lambda i: (i, 0)),
        ],
        out_specs=pl.BlockSpec((1, Q, 8), lambda i: (i, 0, 0)),
        out_shape=jax.ShapeDtypeStruct((NB, Q, 8), jnp.float32),
        interpret=INTERPRET,
    )(qc, kc)

    Csum = C8.sum(axis=0)
    gec = (Csum >= NUM_CANDIDATES)
    j0cnt = gec.sum(axis=1)
    v8 = jnp.asarray(V8, jnp.float32)
    b0v = jnp.where(j0cnt > 0,
                    jnp.take(v8, jnp.clip(j0cnt - 1, 0, 7)),
                    jnp.float32(-34.0))
    f0 = (b0v + 2.0)[:, None]

    E = pl.pallas_call(
        functools.partial(_k3b_body, nkeys=K), grid=(NB,),
        in_specs=[
            pl.BlockSpec((Q, H), lambda i: (0, 0)),
            pl.BlockSpec((KBLK, H), lambda i: (i, 0)),
            pl.BlockSpec((Q, 1), lambda i: (0, 0)),
        ],
        out_specs=pl.BlockSpec((1, Q, 4), lambda i: (i, 0, 0)),
        out_shape=jax.ShapeDtypeStruct((NB, Q, 4), jnp.float32),
        interpret=INTERPRET,
    )(qc, kc, f0)

    Esum = E.sum(axis=0)
    nf = (Esum[:, :3] >= NUM_CANDIDATES).sum(axis=1)
    t = b0v + 2.0 * nf
    cgt = jnp.take_along_axis(Esum, nf[:, None], axis=1)[:, 0]
    quota = NUM_CANDIDATES - cgt

    vcounts = jnp.minimum(
        K - KBLK * jnp.arange(NB), KBLK).astype(jnp.float32)[:, None]
    j0idx = jnp.clip(j0cnt - 1, 0, 7)
    cge_t_coarse = jnp.take_along_axis(
        C8, jnp.broadcast_to(j0idx[None, :, None], (NB, Q, 1)),
        axis=2)[:, :, 0]
    cge_t_coarse = jnp.where((j0cnt > 0)[None, :], cge_t_coarse,
                             jnp.broadcast_to(vcounts, (NB, Q)))
    nfm1 = jnp.clip(nf - 1, 0, 3)
    cge_t_fine = jnp.take_along_axis(
        E, jnp.broadcast_to(nfm1[None, :, None], (NB, Q, 1)),
        axis=2)[:, :, 0]
    cge_t = jnp.where((nf == 0)[None, :], cge_t_coarse, cge_t_fine)
    cge_t2 = jnp.take_along_axis(
        E, jnp.broadcast_to(nf[None, :, None], (NB, Q, 1)), axis=2)[:, :, 0]
    eq_blk = cge_t - cge_t2
    cum = jnp.cumsum(eq_blk, axis=0)
    cum_ex = jnp.concatenate([jnp.zeros((1, Q), jnp.float32), cum[:-1]],
                             axis=0)
    qr = jnp.clip(quota[None, :] - cum_ex, 0.0, float(KBLK))
    qr3 = qr[:, :, None]
    t_in = t[:, None]

    NCH = KBLK // 128
    NCC = NB * NCH
    QB = Q // 2 if Q % 2 == 0 and Q > 512 else Q
    S, Mb = pl.pallas_call(
        functools.partial(_k4_body, nkeys=K), grid=(NB, Q // QB),
        in_specs=[
            pl.BlockSpec((QB, D), lambda i, j: (j, 0)),
            pl.BlockSpec((KBLK, D), lambda i, j: (i, 0)),
            pl.BlockSpec((QB, H), lambda i, j: (j, 0)),
            pl.BlockSpec((KBLK, H), lambda i, j: (i, 0)),
            pl.BlockSpec((QB, 1), lambda i, j: (j, 0)),
            pl.BlockSpec((1, QB, 1), lambda i, j: (i, j, 0)),
        ],
        out_specs=[
            pl.BlockSpec((QB, KBLK), lambda i, j: (j, i)),
            pl.BlockSpec((1, QB, NCH), lambda i, j: (i, j, 0)),
        ],
        out_shape=[
            jax.ShapeDtypeStruct((Q, KP), jnp.float32),
            jax.ShapeDtypeStruct((NB, Q, NCH), jnp.float32),
        ],
        interpret=INTERPRET,
    )(queries, keys_pad, qc, kc, t_in, qr3)

    M2 = jnp.transpose(Mb, (1, 0, 2)).reshape(Q, NCC)
    cval, cidx = pl.pallas_call(
        _topchunk_body, grid=(1,),
        in_specs=[pl.BlockSpec((Q, NCC), lambda i: (0, 0))],
        out_specs=[
            pl.BlockSpec((Q, 16), lambda i: (0, 0)),
            pl.BlockSpec((Q, 16), lambda i: (0, 0)),
        ],
        out_shape=[
            jax.ShapeDtypeStruct((Q, 16), jnp.float32),
            jax.ShapeDtypeStruct((Q, 16), jnp.int32),
        ],
        interpret=INTERPRET,
    )(M2)

    cidx10 = cidx[:, :TOP_K]
    valid10 = (cval[:, :TOP_K] > -1.0e38).astype(jnp.float32)
    c_clip = jnp.maximum(cidx10, 0)
    rows = (jnp.arange(Q, dtype=jnp.int32)[:, None] * NCC
            + c_clip).reshape(-1)
    nrows = Q * TOP_K
    nrows_pad = ((nrows + 2047) // 2048) * 2048
    rows = jnp.concatenate(
        [rows, jnp.zeros((nrows_pad - nrows,), jnp.int32)])
    idx3 = rows.reshape(32, nrows_pad // (32 * 64), 64)
    gidx = (c_clip[:, :, None] * 128
            + jnp.arange(128, dtype=jnp.int32)[None, None, :])
    gidx2 = gidx.reshape(Q, TOP_K * 128)
    valid2 = jnp.broadcast_to(
        valid10[:, :, None], (Q, TOP_K, 128)).reshape(Q, TOP_K * 128)

    S_flat = S.reshape(Q * NCC, 128)
    if INTERPRET:
        gath = jnp.take(S_flat, rows, axis=0)
    else:
        gath = _sc_gather_rows(S_flat, idx3, nrows_pad)
    g2 = gath[:nrows].reshape(Q, TOP_K * 128)

    vals16, idx16 = pl.pallas_call(
        _final_body, grid=(1,),
        in_specs=[
            pl.BlockSpec((Q, TOP_K * 128), lambda i: (0, 0)),
            pl.BlockSpec((Q, TOP_K * 128), lambda i: (0, 0)),
            pl.BlockSpec((Q, TOP_K * 128), lambda i: (0, 0)),
        ],
        out_specs=[
            pl.BlockSpec((Q, 16), lambda i: (0, 0)),
            pl.BlockSpec((Q, 16), lambda i: (0, 0)),
        ],
        out_shape=[
            jax.ShapeDtypeStruct((Q, 16), jnp.float32),
            jax.ShapeDtypeStruct((Q, 16), jnp.int32),
        ],
        interpret=INTERPRET,
    )(g2, gidx2, valid2)

    return vals16[:, :TOP_K], idx16[:, :TOP_K]


# speedup vs baseline: 1.1288x; 1.1288x over previous
import functools

import jax
import jax.numpy as jnp
from jax.experimental import pallas as pl

U = 0.83
NUM_CANDIDATES = 256
TOP_K = 10
KBLK = 2048
NEG = -3.0e38
V8 = (-28.0, -20.0, -12.0, -4.0, 4.0, 12.0, 20.0, 28.0)

def _k1_body(k_ref, out_ref):
    k = k_ref[...]
    r = jnp.sum(k * k, axis=1, keepdims=True)
    m = jnp.max(r, axis=0, keepdims=True)
    out_ref[...] = jnp.broadcast_to(m[None, :, :], out_ref.shape)


def _k2k_body(k_ref, a_ref, inv_ref, out_ref):
    s = inv_ref[0, 0]
    ku = k_ref[...] * s
    n2 = jnp.sum(ku * ku, axis=1, keepdims=True)
    n4 = n2 * n2
    n8 = n4 * n4
    zpad = jnp.zeros((ku.shape[0], 5), jnp.float32)
    p = jnp.concatenate([ku, n2, n4, n8, zpad], axis=1)
    pre = jax.lax.dot_general(
        p, a_ref[...], (((1,), (0,)), ((), ())),
        preferred_element_type=jnp.float32)
    out_ref[...] = jnp.where(pre > 0, 1.0, -1.0).astype(jnp.bfloat16)


def _k2q_body(q_ref, a_ref, inv_ref, out_ref):
    s = inv_ref[0, 0]
    qu = q_ref[...] * s
    halves = jnp.full((qu.shape[0], 3), 0.5, jnp.float32)
    zpad = jnp.zeros((qu.shape[0], 5), jnp.float32)
    p = jnp.concatenate([qu, halves, zpad], axis=1)
    pre = jax.lax.dot_general(
        p, a_ref[...], (((1,), (0,)), ((), ())),
        preferred_element_type=jnp.float32)
    out_ref[...] = jnp.where(pre > 0, 1.0, -1.0).astype(jnp.bfloat16)


def _votes(qc_ref, kc_ref):
    return jax.lax.dot_general(
        qc_ref[...], kc_ref[...], (((1,), (1,)), ((), ())),
        preferred_element_type=jnp.float32)


def _valid_mask(shape, nkeys):
    i = pl.program_id(0)
    kloc = jax.lax.broadcasted_iota(jnp.int32, shape, 1)
    return kloc < (nkeys - i * KBLK)


def _k3a_body(qc_ref, kc_ref, out_ref, *, nkeys):
    v = _votes(qc_ref, kc_ref)
    valid = _valid_mask(v.shape, nkeys)
    cols = []
    for vj in V8:
        c = jnp.sum(jnp.where((v >= vj) & valid, 1.0, 0.0), axis=1,
                    keepdims=True)
        cols.append(c)
    out_ref[...] = jnp.concatenate(cols, axis=1)[None, :, :]


def _k3b_body(qc_ref, kc_ref, f0_ref, out_ref, *, nkeys):
    v = _votes(qc_ref, kc_ref)
    valid = _valid_mask(v.shape, nkeys)
    f0 = f0_ref[...]
    cols = []
    for j in range(4):
        c = jnp.sum(jnp.where((v >= f0 + 2.0 * j) & valid, 1.0, 0.0),
                    axis=1, keepdims=True)
        cols.append(c)
    out_ref[...] = jnp.concatenate(cols, axis=1)[None, :, :]


def _k4_body(q_ref, k_ref, qc_ref, kc_ref, t_ref, qr_ref, s_ref, m_ref,
             *, nkeys):
    v = _votes(qc_ref, kc_ref)
    valid = _valid_mask(v.shape, nkeys)
    t = t_ref[...]
    eq = (v == t) & valid
    gt = (v > t) & valid
    eqf = jnp.where(eq, 1.0, 0.0)
    p = eqf
    sh = 1
    while sh < p.shape[1]:
        shifted = jnp.concatenate(
            [jnp.zeros((p.shape[0], sh), jnp.float32), p[:, :-sh]], axis=1)
        p = p + shifted
        sh *= 2
    ex = p - eqf
    qr = qr_ref[0, :, :]
    sel = gt | (eq & (ex < qr))

    s = jax.lax.dot_general(
        q_ref[...], k_ref[...], (((1,), (1,)), ((), ())),
        preferred_element_type=jnp.float32)
    ms = jnp.where(sel, s, NEG)
    s_ref[...] = ms
    cols = [jnp.max(ms[:, c * 128:(c + 1) * 128], axis=1, keepdims=True)
            for c in range(KBLK // 128)]
    m_ref[...] = jnp.concatenate(cols, axis=1)[None, :, :]


def _topchunk_body(m_ref, cv_ref, ci_ref):
    V = m_ref[...]
    I = jax.lax.broadcasted_iota(jnp.int32, V.shape, 1)
    q_n = V.shape[0]
    vals, idxs = [], []
    for _ in range(TOP_K):
        m = jnp.max(V, axis=1, keepdims=True)
        hit = (V == m) & (m > NEG)
        am = jnp.max(jnp.where(hit, I, -1), axis=1, keepdims=True)
        V = jnp.where(I == am, NEG, V)
        vals.append(m)
        idxs.append(am)
    pad = 16 - TOP_K
    vals.append(jnp.full((q_n, pad), NEG, jnp.float32))
    idxs.append(jnp.full((q_n, pad), -1, jnp.int32))
    cv_ref[...] = jnp.concatenate(vals, axis=1)
    ci_ref[...] = jnp.concatenate(idxs, axis=1)


def _final_body(g_ref, gi_ref, val_ref, v_ref, i_ref):
    cur = jnp.where(val_ref[...] > 0, g_ref[...], NEG)
    I = gi_ref[...]
    q_n = cur.shape[0]
    vals, idxs = [], []
    for _ in range(TOP_K):
        m = jnp.max(cur, axis=1, keepdims=True)
        hit = (cur == m) & (m > NEG)
        am = jnp.max(jnp.where(hit, I, -1), axis=1, keepdims=True)
        cur = jnp.where(I == am, NEG, cur)
        vals.append(m)
        idxs.append(am)
    pad = 16 - TOP_K
    vals.append(jnp.full((q_n, pad), NEG, jnp.float32))
    idxs.append(jnp.full((q_n, pad), -1, jnp.int32))
    v_ref[...] = jnp.concatenate(vals, axis=1)
    i_ref[...] = jnp.concatenate(idxs, axis=1)


def _sc_gather_rows(table, idx3, nrows):
    from jax.experimental.pallas import tpu as pltpu
    from jax.experimental.pallas import tpu_sc as plsc
    from jax import lax
    ch = idx3.shape[1]
    bpw = ch * 64
    mesh = plsc.VectorSubcoreMesh(core_axis_name="c", subcore_axis_name="s")

    def body(tab_ref, idx_ref, out_ref, idx_v, rows_v, sem):
        wid = lax.axis_index("s") * 2 + lax.axis_index("c")
        pltpu.sync_copy(idx_ref.at[wid], idx_v)
        for j in range(ch):
            pltpu.async_copy(tab_ref.at[idx_v.at[j]],
                             rows_v.at[pl.ds(j * 64, 64)], sem).wait()
        pltpu.sync_copy(rows_v, out_ref.at[pl.ds(wid * bpw, bpw)])

    return pl.kernel(
        body,
        out_type=jax.ShapeDtypeStruct((nrows, 128), jnp.float32),
        mesh=mesh,
        scratch_types=[
            pltpu.VMEM((ch, 64), jnp.int32),
            pltpu.VMEM((bpw, 128), jnp.float32),
            pltpu.SemaphoreType.DMA,
        ],
    )(table, idx3)


def kernel(queries, keys, a):
    K, D = keys.shape
    Q = queries.shape[0]
    H = a.shape[1]
    NB = (K + KBLK - 1) // KBLK
    KP = NB * KBLK

    keys_pad = jnp.concatenate(
        [keys, jnp.zeros((KP - K, D), jnp.float32)], axis=0)
    a_pad = jnp.concatenate([a, jnp.zeros((5, H), jnp.float32)], axis=0)

    rmax = pl.pallas_call(
        _k1_body, grid=(NB,),
        in_specs=[pl.BlockSpec((KBLK, D), lambda i: (i, 0))],
        out_specs=pl.BlockSpec((1, 1, 128), lambda i: (i, 0, 0)),
        out_shape=jax.ShapeDtypeStruct((NB, 1, 128), jnp.float32),
    )(keys_pad)
    denom = jnp.sqrt(jnp.max(rmax))
    inv = jnp.full((1, 128), U / denom, jnp.float32)

    kc = pl.pallas_call(
        _k2k_body, grid=(NB,),
        in_specs=[
            pl.BlockSpec((KBLK, D), lambda i: (i, 0)),
            pl.BlockSpec((136, H), lambda i: (0, 0)),
            pl.BlockSpec((1, 128), lambda i: (0, 0)),
        ],
        out_specs=pl.BlockSpec((KBLK, H), lambda i: (i, 0)),
        out_shape=jax.ShapeDtypeStruct((KP, H), jnp.bfloat16),
    )(keys_pad, a_pad, inv)

    qc = pl.pallas_call(
        _k2q_body, grid=(1,),
        in_specs=[
            pl.BlockSpec((Q, D), lambda i: (0, 0)),
            pl.BlockSpec((136, H), lambda i: (0, 0)),
            pl.BlockSpec((1, 128), lambda i: (0, 0)),
        ],
        out_specs=pl.BlockSpec((Q, H), lambda i: (0, 0)),
        out_shape=jax.ShapeDtypeStruct((Q, H), jnp.bfloat16),
    )(queries, a_pad, inv)

    C8 = pl.pallas_call(
        functools.partial(_k3a_body, nkeys=K), grid=(NB,),
        in_specs=[
            pl.BlockSpec((Q, H), lambda i: (0, 0)),
            pl.BlockSpec((KBLK, H), lambda i: (i, 0)),
        ],
        out_specs=pl.BlockSpec((1, Q, 8), lambda i: (i, 0, 0)),
        out_shape=jax.ShapeDtypeStruct((NB, Q, 8), jnp.float32),
    )(qc, kc)

    Csum = C8.sum(axis=0)
    gec = (Csum >= NUM_CANDIDATES)
    j0cnt = gec.sum(axis=1)
    v8 = jnp.asarray(V8, jnp.float32)
    b0v = jnp.where(j0cnt > 0,
                    jnp.take(v8, jnp.clip(j0cnt - 1, 0, 7)),
                    jnp.float32(-34.0))
    f0 = (b0v + 2.0)[:, None]

    E = pl.pallas_call(
        functools.partial(_k3b_body, nkeys=K), grid=(NB,),
        in_specs=[
            pl.BlockSpec((Q, H), lambda i: (0, 0)),
            pl.BlockSpec((KBLK, H), lambda i: (i, 0)),
            pl.BlockSpec((Q, 1), lambda i: (0, 0)),
        ],
        out_specs=pl.BlockSpec((1, Q, 4), lambda i: (i, 0, 0)),
        out_shape=jax.ShapeDtypeStruct((NB, Q, 4), jnp.float32),
    )(qc, kc, f0)

    Esum = E.sum(axis=0)
    nf = (Esum[:, :3] >= NUM_CANDIDATES).sum(axis=1)
    t = b0v + 2.0 * nf
    cgt = jnp.take_along_axis(Esum, nf[:, None], axis=1)[:, 0]
    quota = NUM_CANDIDATES - cgt

    vcounts = jnp.minimum(
        K - KBLK * jnp.arange(NB), KBLK).astype(jnp.float32)[:, None]
    j0idx = jnp.clip(j0cnt - 1, 0, 7)
    cge_t_coarse = jnp.take_along_axis(
        C8, jnp.broadcast_to(j0idx[None, :, None], (NB, Q, 1)),
        axis=2)[:, :, 0]
    cge_t_coarse = jnp.where((j0cnt > 0)[None, :], cge_t_coarse,
                             jnp.broadcast_to(vcounts, (NB, Q)))
    nfm1 = jnp.clip(nf - 1, 0, 3)
    cge_t_fine = jnp.take_along_axis(
        E, jnp.broadcast_to(nfm1[None, :, None], (NB, Q, 1)),
        axis=2)[:, :, 0]
    cge_t = jnp.where((nf == 0)[None, :], cge_t_coarse, cge_t_fine)
    cge_t2 = jnp.take_along_axis(
        E, jnp.broadcast_to(nf[None, :, None], (NB, Q, 1)), axis=2)[:, :, 0]
    eq_blk = cge_t - cge_t2
    cum = jnp.cumsum(eq_blk, axis=0)
    cum_ex = jnp.concatenate([jnp.zeros((1, Q), jnp.float32), cum[:-1]],
                             axis=0)
    qr = jnp.clip(quota[None, :] - cum_ex, 0.0, float(KBLK))
    qr3 = qr[:, :, None]
    t_in = t[:, None]

    NCH = KBLK // 128
    NCC = NB * NCH
    QB = Q // 2 if Q % 2 == 0 and Q > 512 else Q
    S, Mb = pl.pallas_call(
        functools.partial(_k4_body, nkeys=K), grid=(NB, Q // QB),
        in_specs=[
            pl.BlockSpec((QB, D), lambda i, j: (j, 0)),
            pl.BlockSpec((KBLK, D), lambda i, j: (i, 0)),
            pl.BlockSpec((QB, H), lambda i, j: (j, 0)),
            pl.BlockSpec((KBLK, H), lambda i, j: (i, 0)),
            pl.BlockSpec((QB, 1), lambda i, j: (j, 0)),
            pl.BlockSpec((1, QB, 1), lambda i, j: (i, j, 0)),
        ],
        out_specs=[
            pl.BlockSpec((QB, KBLK), lambda i, j: (j, i)),
            pl.BlockSpec((1, QB, NCH), lambda i, j: (i, j, 0)),
        ],
        out_shape=[
            jax.ShapeDtypeStruct((Q, KP), jnp.float32),
            jax.ShapeDtypeStruct((NB, Q, NCH), jnp.float32),
        ],
    )(queries, keys_pad, qc, kc, t_in, qr3)

    M2 = jnp.transpose(Mb, (1, 0, 2)).reshape(Q, NCC)
    cval, cidx = pl.pallas_call(
        _topchunk_body, grid=(1,),
        in_specs=[pl.BlockSpec((Q, NCC), lambda i: (0, 0))],
        out_specs=[
            pl.BlockSpec((Q, 16), lambda i: (0, 0)),
            pl.BlockSpec((Q, 16), lambda i: (0, 0)),
        ],
        out_shape=[
            jax.ShapeDtypeStruct((Q, 16), jnp.float32),
            jax.ShapeDtypeStruct((Q, 16), jnp.int32),
        ],
    )(M2)

    cidx10 = cidx[:, :TOP_K]
    valid10 = (cval[:, :TOP_K] > -1.0e38).astype(jnp.float32)
    c_clip = jnp.maximum(cidx10, 0)
    rows = (jnp.arange(Q, dtype=jnp.int32)[:, None] * NCC
            + c_clip).reshape(-1)
    nrows = Q * TOP_K
    nrows_pad = ((nrows + 2047) // 2048) * 2048
    rows = jnp.concatenate(
        [rows, jnp.zeros((nrows_pad - nrows,), jnp.int32)])
    idx3 = rows.reshape(32, nrows_pad // (32 * 64), 64)
    gidx = (c_clip[:, :, None] * 128
            + jnp.arange(128, dtype=jnp.int32)[None, None, :])
    gidx2 = gidx.reshape(Q, TOP_K * 128)
    valid2 = jnp.broadcast_to(
        valid10[:, :, None], (Q, TOP_K, 128)).reshape(Q, TOP_K * 128)

    S_flat = S.reshape(Q * NCC, 128)
    gath = _sc_gather_rows(S_flat, idx3, nrows_pad)
    g2 = gath[:nrows].reshape(Q, TOP_K * 128)

    vals16, idx16 = pl.pallas_call(
        _final_body, grid=(1,),
        in_specs=[
            pl.BlockSpec((Q, TOP_K * 128), lambda i: (0, 0)),
            pl.BlockSpec((Q, TOP_K * 128), lambda i: (0, 0)),
            pl.BlockSpec((Q, TOP_K * 128), lambda i: (0, 0)),
        ],
        out_specs=[
            pl.BlockSpec((Q, 16), lambda i: (0, 0)),
            pl.BlockSpec((Q, 16), lambda i: (0, 0)),
        ],
        out_shape=[
            jax.ShapeDtypeStruct((Q, 16), jnp.float32),
            jax.ShapeDtypeStruct((Q, 16), jnp.int32),
        ],
    )(g2, gidx2, valid2)

    return vals16[:, :TOP_K], idx16[:, :TOP_K]
